# Initial kernel scaffold; baseline (speedup 1.0000x reference)
import jax
import jax.numpy as jnp
from jax.experimental import pallas as pl


def kernel(x, ids, cond, sid, quant_tables, cond_table, ch_table, sub_table, sep_table, ts_table):
    raise NotImplementedError("write your pallas kernel here")


# speedup vs baseline: 3.4656x; 3.4656x over previous
import functools

import jax
import jax.numpy as jnp
from jax import lax
from jax.experimental import pallas as pl
from jax.experimental.pallas import tpu as pltpu, tpu_sc as plsc

B, C, T, E = 8, 32, 2048, 64
QL, NCLS, SUBJ, SR = 256, 40, 64, 2048

GPT = C + 1
ROWS_PER_B = 1 + T * GPT
NROWS = B * ROWS_PER_B
NWORK = 32
TPW = T // 4
CT = 16
CROWS = CT * GPT
NCHUNK = TPW // CT
GJ, GW = 6, 88
SEP_IDX = C * QL


def _sc_body(x2d, ids_h, condv_h, sidv_h, qt_h, ctab_h, chtab_h, subtab_h,
             sep_h, ts_h, out_h,
             idsv, sidv, condv, cidx, subrows, condrows, chrows, basec,
             sepv, xch, tsch, fidx, dst, sem):
    wid = lax.axis_index("c") * 16 + lax.axis_index("s")
    b = wid // 4
    tp = wid % 4

    pltpu.sync_copy(ids_h, idsv)
    pltpu.sync_copy(sidv_h, sidv)
    pltpu.sync_copy(condv_h, condv)
    pltpu.sync_copy(sep_h, sepv)

    pltpu.async_copy(subtab_h.at[sidv], subrows, sem).wait()

    cv = condv[pl.ds(0, 16)]
    cidx[pl.ds(0, 16)] = jnp.where(cv > 0, cv, NCLS)
    pltpu.async_copy(ctab_h.at[cidx], condrows, sem).wait()
    pltpu.async_copy(chtab_h.at[idsv], chrows, sem).wait()

    for l in range(4):
        sl = pl.ds(l * 16, 16)
        bseg = subrows[b, sl] + condrows[b, sl]
        for c in range(C):
            basec[c, sl] = chrows[c, sl] + bseg

    @pl.when(tp == 0)
    def _():
        pltpu.sync_copy(sepv, out_h.at[pl.ds(b * ROWS_PER_B, 1)])

    lanes = lax.iota(jnp.int32, 16)

    def chunk(g, carry):
        t0 = tp * TPW + g * CT
        r0 = b * ROWS_PER_B + 1 + t0 * GPT

        pltpu.sync_copy(x2d.at[pl.ds(b * C, C), pl.ds(t0, CT)], xch)
        pltpu.sync_copy(ts_h.at[pl.ds(t0, CT)], tsch)

        def cbody(c, carry):
            pos = lanes * GPT + c
            vals = xch[c, pl.ds(0, 16)] + c * QL
            plsc.store_scatter(fidx, [pos], vals)
            return carry
        lax.fori_loop(0, C, cbody, 0)
        pos = lanes * GPT + C
        plsc.store_scatter(fidx, [pos], jnp.full((16,), SEP_IDX, jnp.int32))

        descs = [pltpu.async_copy(qt_h.at[fidx.at[pl.ds(j * GW, GW)]],
                                  dst.at[pl.ds(j * GW, GW)], sem)
                 for j in range(GJ)]
        for d in descs:
            d.wait()

        def kbody(k, carry):
            tseg = [tsch[k, pl.ds(l * 16, 16)] for l in range(4)]

            def cadd(c, carry):
                row = k * GPT + c
                for l in range(4):
                    sl = pl.ds(l * 16, 16)
                    plsc.addupdate(dst.at[row, sl], basec[c, sl] + tseg[l])
                return carry
            lax.fori_loop(0, C, cadd, 0)
            srow = k * GPT + C
            for l in range(4):
                sl = pl.ds(l * 16, 16)
                dst[srow, sl] = sepv[0, sl]
            return carry
        lax.fori_loop(0, CT, kbody, 0)

        pltpu.sync_copy(dst, out_h.at[pl.ds(r0, CROWS)])
        return carry

    lax.fori_loop(0, NCHUNK, chunk, 0)


def kernel(x, ids, cond, sid, quant_tables, cond_table, ch_table, sub_table,
           sep_table, ts_table):
    x2d = x.astype(jnp.int32).reshape(B * C, T)
    ids32 = ids.astype(jnp.int32)
    condv = jnp.pad(cond.astype(jnp.int32).reshape(B), (0, 16 - B))
    sidv = jnp.pad(sid.astype(jnp.int32).reshape(B), (0, 16 - B))
    qt_ext = jnp.concatenate(
        [quant_tables.reshape(C * QL, E), jnp.zeros((8, E), jnp.float32)])
    ctab_ext = jnp.concatenate([cond_table, jnp.zeros((8, E), jnp.float32)])

    run = pl.kernel(
        _sc_body,
        out_type=jax.ShapeDtypeStruct((NROWS, E), jnp.float32),
        mesh=plsc.VectorSubcoreMesh(core_axis_name="c", subcore_axis_name="s"),
        compiler_params=pltpu.CompilerParams(use_tc_tiling_on_sc=False,
                                             needs_layout_passes=False),
        scratch_types=[
            pltpu.VMEM((C,), jnp.int32),
            pltpu.VMEM((16,), jnp.int32),
            pltpu.VMEM((16,), jnp.int32),
            pltpu.VMEM((16,), jnp.int32),
            pltpu.VMEM((16, E), jnp.float32),
            pltpu.VMEM((16, E), jnp.float32),
            pltpu.VMEM((C, E), jnp.float32),
            pltpu.VMEM((C, E), jnp.float32),
            pltpu.VMEM((1, E), jnp.float32),
            pltpu.VMEM((C, CT), jnp.int32),
            pltpu.VMEM((CT, E), jnp.float32),
            pltpu.VMEM((CROWS,), jnp.int32),
            pltpu.VMEM((CROWS, E), jnp.float32),
            pltpu.SemaphoreType.DMA,
        ],
    )
    out_flat = run(x2d, ids32, condv, sidv, qt_ext, ctab_ext, ch_table,
                   sub_table, sep_table, ts_table)
    return out_flat.reshape(B, ROWS_PER_B, E)
